# baseline (speedup 1.0000x reference)
import functools

import jax
import jax.numpy as jnp
import numpy as np
from jax import lax
from jax.experimental import pallas as pl
from jax.experimental.pallas import tpu as pltpu
from jax.experimental.pallas import tpu_sc as plsc

N = 10000
E = 320000
C = 128
H = 8
NB = 4
L = 3
EPS = 1e-5

NPAD = 10240
RPT = NPAD // 16
KD = 128
KM = 64
CH_MAIN = 160
CH_DEG = 80
EPAD = CH_MAIN * KM * 32
F = 192
BR = 640
GRID = NPAD // BR

_mesh = plsc.VectorSubcoreMesh(core_axis_name="c", subcore_axis_name="s")


@functools.partial(
    pl.kernel,
    out_type=jax.ShapeDtypeStruct((32, NPAD), jnp.float32),
    mesh=_mesh,
    scratch_types=[
        pltpu.VMEM((CH_DEG, KD), jnp.int32),
        pltpu.VMEM((NPAD,), jnp.float32),
    ],
    compiler_params=pltpu.CompilerParams(needs_layout_passes=False),
)
def _deg_pass(cols_hbm, out_hbm, idx_v, acc_v):
    c = lax.axis_index("c")
    s = lax.axis_index("s")
    w = c * 16 + s
    pltpu.sync_copy(cols_hbm.at[w], idx_v)

    def zbody(j, carry):
        acc_v[pl.ds(j * 16, 16)] = jnp.zeros((16,), jnp.float32)
        return carry

    lax.fori_loop(0, NPAD // 16, zbody, 0)
    ones = jnp.ones((16,), jnp.float32)

    def body(j, carry):
        for t in range(KD // 16):
            idx16 = idx_v[j, pl.ds(t * 16, 16)]
            plsc.addupdate_scatter(acc_v, [idx16], ones)
        return carry

    lax.fori_loop(0, CH_DEG, body, 0)
    pltpu.sync_copy(acc_v, out_hbm.at[w])


@functools.partial(
    pl.kernel,
    out_type=jax.ShapeDtypeStruct((2, NPAD, C), jnp.float32),
    mesh=_mesh,
    scratch_types=[
        pltpu.VMEM((CH_MAIN // 4, KM), jnp.int32),
        pltpu.VMEM((CH_MAIN // 4, KM), jnp.int32),
        pltpu.VMEM((KM, C), jnp.float32),
        pltpu.VMEM((KM, C), jnp.float32),
        pltpu.VMEM((KM, C), jnp.float32),
        pltpu.VMEM((KM, C), jnp.float32),
        pltpu.SemaphoreType.DMA,
        pltpu.SemaphoreType.DMA,
        pltpu.SemaphoreType.DMA,
        pltpu.SemaphoreType.DMA,
        pltpu.VMEM_SHARED((NPAD, C), jnp.float32),
    ],
)
def _scatter_pass(rows_hbm, cols_hbm, table_hbm, out_hbm,
                  rid_v, cid_v, buf0, buf1, buf2, buf3,
                  sem0, sem1, sem2, sem3, acc_sh):
    c = lax.axis_index("c")
    s = lax.axis_index("s")
    segch = CH_MAIN // 4
    pltpu.sync_copy(table_hbm.at[pl.ds(c * NPAD + s * RPT, RPT)],
                    acc_sh.at[pl.ds(s * RPT, RPT)])
    plsc.subcore_barrier()

    rings = ((buf0, sem0), (buf1, sem1), (buf2, sem2), (buf3, sem3))

    def body(j, carry):
        j4 = j * 4
        for b, (buf, sem) in enumerate(rings):
            pltpu.make_async_copy(table_hbm.at[rid_v.at[j4 + b]], buf, sem).wait()
            pltpu.sync_copy(buf, acc_sh.at[cid_v.at[j4 + b]], add=True)
            pltpu.async_copy(table_hbm.at[rid_v.at[j4 + b + 4]], buf, sem)
        return carry

    for seg in range(4):
        pltpu.sync_copy(rows_hbm.at[c, s, pl.ds(seg * segch, segch)], rid_v)
        pltpu.sync_copy(cols_hbm.at[c, s, pl.ds(seg * segch, segch)], cid_v)
        for b, (buf, sem) in enumerate(rings):
            pltpu.async_copy(table_hbm.at[rid_v.at[b]], buf, sem)
        lax.fori_loop(0, segch // 4 - 1, body, 0)
        last = segch - 4
        for b, (buf, sem) in enumerate(rings):
            pltpu.make_async_copy(table_hbm.at[rid_v.at[last + b]], buf, sem).wait()
            pltpu.sync_copy(buf, acc_sh.at[cid_v.at[last + b]], add=True)
    plsc.subcore_barrier()
    pltpu.sync_copy(acc_sh.at[pl.ds(s * RPT, RPT)], out_hbm.at[c, pl.ds(s * RPT, RPT)])


def _prep_body(deg_ref, data_ref, xs_ref, dis_ref):
    dp = deg_ref[...]
    dis = lax.rsqrt(jnp.sum(dp, axis=0) + 1.0)
    d2 = dis[:, None]
    xs = data_ref[...] * d2
    xs_ref[...] = jnp.stack([xs, xs], axis=0)
    dis_ref[...] = d2


def _h_body(acc_ref, dis_ref, data_ref, wbt_ref, wc_ref, bc_ref, rm_ref,
            tm_ref, cb_ref, h_ref, psum_ref, psumsq_ref):
    i = pl.program_id(0)
    a = acc_ref[...]
    d2 = dis_ref[...]
    x = data_ref[...]
    sagg = a[0] + a[1] - x * d2
    agg = d2 * jnp.dot(sagg, wbt_ref[...], preferred_element_type=jnp.float32)
    w = jnp.dot(x, wc_ref[...], preferred_element_type=jnp.float32) + bc_ref[...]
    h = None
    for b in range(NB):
        t = (jnp.dot(w, rm_ref[b], preferred_element_type=jnp.float32)
             * jnp.dot(agg, tm_ref[b], preferred_element_type=jnp.float32))
        h = t if h is None else h + t
    h = h + cb_ref[...]
    h_ref[...] = h
    rowid = lax.broadcasted_iota(jnp.int32, (BR, 1), 0) + i * BR
    hm = jnp.where(rowid < N, h, 0.0)
    psum_ref[...] = jnp.sum(hm, axis=0, keepdims=True)[None]
    psumsq_ref[...] = jnp.sum(hm * hm, axis=0, keepdims=True)[None]


def _out_body(h_ref, data_ref, scale_ref, shift_ref, fcd_ref, fch_ref, fcb_ref,
              out_ref):
    hn = jnp.maximum(h_ref[...] * scale_ref[...] + shift_ref[...], 0.0)
    out_ref[...] = (jnp.dot(data_ref[...], fcd_ref[...], preferred_element_type=jnp.float32)
                    + jnp.dot(hn, fch_ref[...], preferred_element_type=jnp.float32)
                    + fcb_ref[...])


def _full(shape):
    return pl.BlockSpec(shape, lambda i: tuple(0 for _ in shape))


def _make_masks():
    rm = np.zeros((NB, 96, 384), np.float32)
    tm = np.zeros((NB, F, 384), np.float32)
    for i in range(L):
        for b in range(NB):
            for h in range(H):
                rm[b, 32 * i + 4 * h + b, 128 * i + 16 * h:128 * i + 16 * h + 16] = 1.0
                for cc in range(16):
                    tm[b, 64 * i + 16 * b + cc, 128 * i + 16 * h + cc] = 1.0
    return jnp.asarray(rm), jnp.asarray(tm)


def kernel(data, edge_index, Wb0, Wc0, bc0, cb0, g0, b0, Wb1, Wc1, bc1, cb1,
           g1, b1, Wb2, Wc2, bc2, cb2, g2, b2, fcW, fcb):
    f32 = jnp.float32
    row = edge_index[0]
    col = edge_index[1]
    pad = EPAD - E
    rowp = jnp.concatenate([row, jnp.zeros((pad,), jnp.int32)])
    colp = jnp.concatenate([col, jnp.full((pad,), N, jnp.int32)])
    cols_deg = colp.reshape(32, CH_DEG, KD)
    rows_il = rowp.reshape(-1, 2, KM).transpose(1, 0, 2)
    cols_il = colp.reshape(-1, 2, KM).transpose(1, 0, 2)
    rows_main = (rows_il.reshape(2, 16, CH_MAIN, KM)
                 + jnp.arange(2, dtype=jnp.int32).reshape(2, 1, 1, 1) * NPAD)
    cols_main = cols_il.reshape(2, 16, CH_MAIN, KM)
    data_pad = jnp.pad(data, ((0, NPAD - N), (0, 0)))

    WbT = jnp.concatenate([Wb0, Wb1, Wb2], axis=0).T
    WcT = jnp.concatenate([Wc0, Wc1, Wc2], axis=0).T
    bc_all = jnp.concatenate([bc0, bc1, bc2])[None, :]
    cb_all = jnp.concatenate([cb0, cb1, cb2])[None, :]
    rm, tm = _make_masks()

    deg_parts = _deg_pass(cols_deg)

    xs, dis = pl.pallas_call(
        _prep_body,
        grid=(GRID,),
        in_specs=[
            pl.BlockSpec((32, BR), lambda i: (0, i)),
            pl.BlockSpec((BR, C), lambda i: (i, 0)),
        ],
        out_specs=[
            pl.BlockSpec((2, BR, C), lambda i: (0, i, 0)),
            pl.BlockSpec((BR, 1), lambda i: (i, 0)),
        ],
        out_shape=[
            jax.ShapeDtypeStruct((2, NPAD, C), f32),
            jax.ShapeDtypeStruct((NPAD, 1), f32),
        ],
    )(deg_parts, data_pad)

    acc = _scatter_pass(rows_main, cols_main, xs.reshape(2 * NPAD, C))

    h_all, psum, psumsq = pl.pallas_call(
        _h_body,
        grid=(GRID,),
        in_specs=[
            pl.BlockSpec((2, BR, C), lambda i: (0, i, 0)),
            pl.BlockSpec((BR, 1), lambda i: (i, 0)),
            pl.BlockSpec((BR, C), lambda i: (i, 0)),
            _full((C, F)),
            _full((C, 96)),
            _full((1, 96)),
            _full((NB, 96, 384)),
            _full((NB, F, 384)),
            _full((1, 384)),
        ],
        out_specs=[
            pl.BlockSpec((BR, 384), lambda i: (i, 0)),
            pl.BlockSpec((1, 1, 384), lambda i: (i, 0, 0)),
            pl.BlockSpec((1, 1, 384), lambda i: (i, 0, 0)),
        ],
        out_shape=[
            jax.ShapeDtypeStruct((NPAD, 384), f32),
            jax.ShapeDtypeStruct((GRID, 1, 384), f32),
            jax.ShapeDtypeStruct((GRID, 1, 384), f32),
        ],
    )(acc, dis, data_pad, WbT, WcT, bc_all, rm, tm, cb_all)

    mean = jnp.sum(psum[:, 0, :], axis=0) / N
    var = jnp.sum(psumsq[:, 0, :], axis=0) / N - mean * mean
    istd = lax.rsqrt(var + EPS)
    g_all = jnp.concatenate([g0, g1, g2])
    b_all = jnp.concatenate([b0, b1, b2])
    scale = (g_all * istd)[None, :]
    shift = (b_all - mean * g_all * istd)[None, :]

    out = pl.pallas_call(
        _out_body,
        grid=(GRID,),
        in_specs=[
            pl.BlockSpec((BR, 384), lambda i: (i, 0)),
            pl.BlockSpec((BR, C), lambda i: (i, 0)),
            _full((1, 384)),
            _full((1, 384)),
            _full((C, C)),
            _full((384, C)),
            _full((1, C)),
        ],
        out_specs=pl.BlockSpec((BR, C), lambda i: (i, 0)),
        out_shape=jax.ShapeDtypeStruct((N, C), f32),
    )(h_all, data_pad, scale, shift, fcW[:, :C].T, fcW[:, C:].T, fcb[None, :])

    return out


# speedup vs baseline: 1.0467x; 1.0467x over previous
import functools

import jax
import jax.numpy as jnp
import numpy as np
from jax import lax
from jax.experimental import pallas as pl
from jax.experimental.pallas import tpu as pltpu
from jax.experimental.pallas import tpu_sc as plsc

N = 10000
E = 320000
C = 128
H = 8
NB = 4
L = 3
EPS = 1e-5

NPAD = 10240
RPT = NPAD // 16
K = 128
CH_MAIN = 80
CH_DEG = 80
EPAD = CH_MAIN * K * 32
F = 192
BR = 640
GRID = NPAD // BR

_mesh = plsc.VectorSubcoreMesh(core_axis_name="c", subcore_axis_name="s")


@functools.partial(
    pl.kernel,
    out_type=jax.ShapeDtypeStruct((32, NPAD), jnp.float32),
    mesh=_mesh,
    scratch_types=[
        pltpu.VMEM((CH_DEG, K), jnp.int32),
        pltpu.VMEM((NPAD,), jnp.float32),
    ],
    compiler_params=pltpu.CompilerParams(needs_layout_passes=False),
)
def _deg_pass(cols_hbm, out_hbm, idx_v, acc_v):
    c = lax.axis_index("c")
    s = lax.axis_index("s")
    w = c * 16 + s
    pltpu.sync_copy(cols_hbm.at[w], idx_v)

    def zbody(j, carry):
        acc_v[pl.ds(j * 16, 16)] = jnp.zeros((16,), jnp.float32)
        return carry

    lax.fori_loop(0, NPAD // 16, zbody, 0)
    ones = jnp.ones((16,), jnp.float32)

    def body(j, carry):
        for t in range(K // 16):
            idx16 = idx_v[j, pl.ds(t * 16, 16)]
            plsc.addupdate_scatter(acc_v, [idx16], ones)
        return carry

    lax.fori_loop(0, CH_DEG, body, 0)
    pltpu.sync_copy(acc_v, out_hbm.at[w])


@functools.partial(
    pl.kernel,
    out_type=jax.ShapeDtypeStruct((2, NPAD, C), jnp.float32),
    mesh=_mesh,
    scratch_types=[
        pltpu.VMEM((CH_MAIN // 2, K), jnp.int32),
        pltpu.VMEM((CH_MAIN // 2, K), jnp.int32),
        pltpu.VMEM((K, C), jnp.float32),
        pltpu.VMEM((K, C), jnp.float32),
        pltpu.SemaphoreType.DMA,
        pltpu.SemaphoreType.DMA,
        pltpu.VMEM_SHARED((NPAD, C), jnp.float32),
    ],
)
def _scatter_pass(rows_hbm, cols_hbm, table_hbm, out_hbm,
                  rid_v, cid_v, buf0, buf1, sem0, sem1, acc_sh):
    c = lax.axis_index("c")
    s = lax.axis_index("s")
    segch = CH_MAIN // 2
    pltpu.sync_copy(table_hbm.at[pl.ds(c * NPAD + s * RPT, RPT)],
                    acc_sh.at[pl.ds(s * RPT, RPT)])
    plsc.subcore_barrier()

    def body(j, carry):
        j2 = j * 2
        pltpu.make_async_copy(table_hbm.at[rid_v.at[j2]], buf0, sem0).wait()
        pltpu.sync_copy(buf0, acc_sh.at[cid_v.at[j2]], add=True)
        pltpu.async_copy(table_hbm.at[rid_v.at[j2 + 2]], buf0, sem0)
        pltpu.make_async_copy(table_hbm.at[rid_v.at[j2 + 1]], buf1, sem1).wait()
        pltpu.sync_copy(buf1, acc_sh.at[cid_v.at[j2 + 1]], add=True)
        pltpu.async_copy(table_hbm.at[rid_v.at[j2 + 3]], buf1, sem1)
        return carry

    for seg in range(2):
        pltpu.sync_copy(rows_hbm.at[c, s, pl.ds(seg * segch, segch)], rid_v)
        pltpu.sync_copy(cols_hbm.at[c, s, pl.ds(seg * segch, segch)], cid_v)
        pltpu.async_copy(table_hbm.at[rid_v.at[0]], buf0, sem0)
        pltpu.async_copy(table_hbm.at[rid_v.at[1]], buf1, sem1)
        lax.fori_loop(0, segch // 2 - 1, body, 0)
        last = segch - 2
        pltpu.make_async_copy(table_hbm.at[rid_v.at[last]], buf0, sem0).wait()
        pltpu.sync_copy(buf0, acc_sh.at[cid_v.at[last]], add=True)
        pltpu.make_async_copy(table_hbm.at[rid_v.at[last + 1]], buf1, sem1).wait()
        pltpu.sync_copy(buf1, acc_sh.at[cid_v.at[last + 1]], add=True)
    plsc.subcore_barrier()
    pltpu.sync_copy(acc_sh.at[pl.ds(s * RPT, RPT)], out_hbm.at[c, pl.ds(s * RPT, RPT)])


def _prep_body(deg_ref, data_ref, xs_ref, dis_ref):
    dp = deg_ref[...]
    dis = lax.rsqrt(jnp.sum(dp, axis=0) + 1.0)
    d2 = dis[:, None]
    xs = data_ref[...] * d2
    xs_ref[...] = jnp.stack([xs, xs], axis=0)
    dis_ref[...] = d2


def _h_body(acc_ref, dis_ref, data_ref, wbt_ref, wc_ref, bc_ref, rm_ref,
            tm_ref, cb_ref, h_ref, psum_ref, psumsq_ref):
    i = pl.program_id(0)
    a = acc_ref[...]
    d2 = dis_ref[...]
    x = data_ref[...]
    sagg = a[0] + a[1] - x * d2
    agg = d2 * jnp.dot(sagg, wbt_ref[...], preferred_element_type=jnp.float32)
    w = jnp.dot(x, wc_ref[...], preferred_element_type=jnp.float32) + bc_ref[...]
    h = None
    for b in range(NB):
        t = (jnp.dot(w, rm_ref[b], preferred_element_type=jnp.float32)
             * jnp.dot(agg, tm_ref[b], preferred_element_type=jnp.float32))
        h = t if h is None else h + t
    h = h + cb_ref[...]
    h_ref[...] = h
    rowid = lax.broadcasted_iota(jnp.int32, (BR, 1), 0) + i * BR
    hm = jnp.where(rowid < N, h, 0.0)
    psum_ref[...] = jnp.sum(hm, axis=0, keepdims=True)[None]
    psumsq_ref[...] = jnp.sum(hm * hm, axis=0, keepdims=True)[None]


def _out_body(h_ref, data_ref, scale_ref, shift_ref, fcd_ref, fch_ref, fcb_ref,
              out_ref):
    hn = jnp.maximum(h_ref[...] * scale_ref[...] + shift_ref[...], 0.0)
    out_ref[...] = (jnp.dot(data_ref[...], fcd_ref[...], preferred_element_type=jnp.float32)
                    + jnp.dot(hn, fch_ref[...], preferred_element_type=jnp.float32)
                    + fcb_ref[...])


def _full(shape):
    return pl.BlockSpec(shape, lambda i: tuple(0 for _ in shape))


def _make_masks():
    rm = np.zeros((NB, 96, 384), np.float32)
    tm = np.zeros((NB, F, 384), np.float32)
    for i in range(L):
        for b in range(NB):
            for h in range(H):
                rm[b, 32 * i + 4 * h + b, 128 * i + 16 * h:128 * i + 16 * h + 16] = 1.0
                for cc in range(16):
                    tm[b, 64 * i + 16 * b + cc, 128 * i + 16 * h + cc] = 1.0
    return jnp.asarray(rm), jnp.asarray(tm)


def kernel(data, edge_index, Wb0, Wc0, bc0, cb0, g0, b0, Wb1, Wc1, bc1, cb1,
           g1, b1, Wb2, Wc2, bc2, cb2, g2, b2, fcW, fcb):
    f32 = jnp.float32
    row = edge_index[0]
    col = edge_index[1]
    pad = EPAD - E
    rowp = jnp.concatenate([row, jnp.zeros((pad,), jnp.int32)])
    colp = jnp.concatenate([col, jnp.full((pad,), N, jnp.int32)])
    cols_deg = colp.reshape(32, CH_DEG, K)
    rows_il = rowp.reshape(-1, 2, K).transpose(1, 0, 2)
    cols_il = colp.reshape(-1, 2, K).transpose(1, 0, 2)
    rows_main = (rows_il.reshape(2, 16, CH_MAIN, K)
                 + jnp.arange(2, dtype=jnp.int32).reshape(2, 1, 1, 1) * NPAD)
    cols_main = cols_il.reshape(2, 16, CH_MAIN, K)
    data_pad = jnp.pad(data, ((0, NPAD - N), (0, 0)))

    WbT = jnp.concatenate([Wb0, Wb1, Wb2], axis=0).T
    WcT = jnp.concatenate([Wc0, Wc1, Wc2], axis=0).T
    bc_all = jnp.concatenate([bc0, bc1, bc2])[None, :]
    cb_all = jnp.concatenate([cb0, cb1, cb2])[None, :]
    rm, tm = _make_masks()

    deg_parts = _deg_pass(cols_deg)

    xs, dis = pl.pallas_call(
        _prep_body,
        grid=(GRID,),
        in_specs=[
            pl.BlockSpec((32, BR), lambda i: (0, i)),
            pl.BlockSpec((BR, C), lambda i: (i, 0)),
        ],
        out_specs=[
            pl.BlockSpec((2, BR, C), lambda i: (0, i, 0)),
            pl.BlockSpec((BR, 1), lambda i: (i, 0)),
        ],
        out_shape=[
            jax.ShapeDtypeStruct((2, NPAD, C), f32),
            jax.ShapeDtypeStruct((NPAD, 1), f32),
        ],
    )(deg_parts, data_pad)

    acc = _scatter_pass(rows_main, cols_main, xs.reshape(2 * NPAD, C))

    h_all, psum, psumsq = pl.pallas_call(
        _h_body,
        grid=(GRID,),
        in_specs=[
            pl.BlockSpec((2, BR, C), lambda i: (0, i, 0)),
            pl.BlockSpec((BR, 1), lambda i: (i, 0)),
            pl.BlockSpec((BR, C), lambda i: (i, 0)),
            _full((C, F)),
            _full((C, 96)),
            _full((1, 96)),
            _full((NB, 96, 384)),
            _full((NB, F, 384)),
            _full((1, 384)),
        ],
        out_specs=[
            pl.BlockSpec((BR, 384), lambda i: (i, 0)),
            pl.BlockSpec((1, 1, 384), lambda i: (i, 0, 0)),
            pl.BlockSpec((1, 1, 384), lambda i: (i, 0, 0)),
        ],
        out_shape=[
            jax.ShapeDtypeStruct((NPAD, 384), f32),
            jax.ShapeDtypeStruct((GRID, 1, 384), f32),
            jax.ShapeDtypeStruct((GRID, 1, 384), f32),
        ],
    )(acc, dis, data_pad, WbT, WcT, bc_all, rm, tm, cb_all)

    mean = jnp.sum(psum[:, 0, :], axis=0) / N
    var = jnp.sum(psumsq[:, 0, :], axis=0) / N - mean * mean
    istd = lax.rsqrt(var + EPS)
    g_all = jnp.concatenate([g0, g1, g2])
    b_all = jnp.concatenate([b0, b1, b2])
    scale = (g_all * istd)[None, :]
    shift = (b_all - mean * g_all * istd)[None, :]

    out = pl.pallas_call(
        _out_body,
        grid=(GRID,),
        in_specs=[
            pl.BlockSpec((BR, 384), lambda i: (i, 0)),
            pl.BlockSpec((BR, C), lambda i: (i, 0)),
            _full((1, 384)),
            _full((1, 384)),
            _full((C, C)),
            _full((384, C)),
            _full((1, C)),
        ],
        out_specs=pl.BlockSpec((BR, C), lambda i: (i, 0)),
        out_shape=jax.ShapeDtypeStruct((N, C), f32),
    )(h_all, data_pad, scale, shift, fcW[:, :C].T, fcW[:, C:].T, fcb[None, :])

    return out
